# baseline (speedup 1.0000x reference)
import functools

import jax
import jax.numpy as jnp
from jax import lax
from jax.experimental import pallas as pl
from jax.experimental.pallas import tpu as pltpu
from jax.experimental.pallas import tpu_sc as plsc

EMBED_K = 32
BATCH = 16384
NROWS = 1000000
NC = 2
NS = 16
LANES = 16
NW = NC * NS
B_PER_W = BATCH // NW
GROUPS = B_PER_W // LANES

BLK = 32768
NBLK = (NROWS + BLK - 1) // BLK


def _matvec_body(w_ref, t_ref, o_ref):
    r = jax.lax.dot_general(w_ref[...], t_ref[...], (((1,), (0,)), ((), ())),
                            preferred_element_type=jnp.float32)
    o_ref[...] = r[0]


def _matvec(table_t, wvec):
    return pl.pallas_call(
        _matvec_body,
        grid=(NBLK,),
        in_specs=[
            pl.BlockSpec((1, EMBED_K), lambda b: (0, 0)),
            pl.BlockSpec((EMBED_K, BLK), lambda b: (0, b)),
        ],
        out_specs=pl.BlockSpec((BLK,), lambda b: (b,)),
        out_shape=jax.ShapeDtypeStruct((NROWS,), jnp.float32),
    )(wvec, table_t)


def _granule_fire(idx_v, src_hbm, dst_v, sem):
    def fire(g, carry):
        vec = idx_v[pl.ds(g * LANES, LANES)]
        for j in range(LANES):
            r = g * LANES + j
            a = pl.multiple_of((vec[j] >> 3) << 3, 8)
            pltpu.async_copy(src_hbm.at[pl.ds(a, 8)],
                             dst_v.at[pl.ds(r * 8, 8)], sem)
        return carry

    lax.fori_loop(0, GROUPS, fire, 0, unroll=False)


def _gather_p_body(uidx_hbm, p_hbm, pu_hbm, uidx_v, gr_v, pu_v, sem):
    wid = lax.axis_index("s") * NC + lax.axis_index("c")
    base = wid * B_PER_W

    pltpu.sync_copy(uidx_hbm.at[pl.ds(base, B_PER_W)], uidx_v)
    _granule_fire(uidx_v, p_hbm, gr_v, sem)
    pltpu.make_async_copy(p_hbm.at[pl.ds(0, 8 * B_PER_W)], gr_v, sem).wait()

    lane8 = lax.iota(jnp.int32, LANES) * 8

    def group(g, carry):
        off = uidx_v[pl.ds(g * LANES, LANES)] & 7
        pu_v[pl.ds(g * LANES, LANES)] = plsc.load_gather(
            gr_v, [g * (LANES * 8) + lane8 + off])
        return carry

    lax.fori_loop(0, GROUPS, group, 0, unroll=False)
    pltpu.sync_copy(pu_v, pu_hbm.at[pl.ds(base, B_PER_W)])


def _gather_q_body(iidx_hbm, q_hbm, pu_hbm, wb_hbm, out_hbm,
                   iidx_v, gr_v, pu_v, wb_v, out_v, sem):
    wid = lax.axis_index("s") * NC + lax.axis_index("c")
    base = wid * B_PER_W

    pltpu.sync_copy(iidx_hbm.at[pl.ds(base, B_PER_W)], iidx_v)
    pltpu.sync_copy(pu_hbm.at[pl.ds(base, B_PER_W)], pu_v)
    pltpu.sync_copy(wb_hbm, wb_v)
    _granule_fire(iidx_v, q_hbm, gr_v, sem)
    pltpu.make_async_copy(q_hbm.at[pl.ds(0, 8 * B_PER_W)], gr_v, sem).wait()

    bias = wb_v[pl.ds(0, LANES)][0]
    lane8 = lax.iota(jnp.int32, LANES) * 8

    def group(g, carry):
        off = iidx_v[pl.ds(g * LANES, LANES)] & 7
        qv = plsc.load_gather(gr_v, [g * (LANES * 8) + lane8 + off])
        z = pu_v[pl.ds(g * LANES, LANES)] + qv + bias
        out_v[pl.ds(g * LANES, LANES)] = 1.0 / (1.0 + jnp.exp(-z))
        return carry

    lax.fori_loop(0, GROUPS, group, 0, unroll=False)
    pltpu.sync_copy(out_v, out_hbm.at[pl.ds(base, B_PER_W)])


@functools.partial(jax.jit, static_argnames=())
def kernel(x, W, H, lin_w, lin_b):
    uidx = x[:, 0].astype(jnp.int32)
    iidx = x[:, 1].astype(jnp.int32)
    wu = lin_w[:, :EMBED_K]
    wv = lin_w[:, EMBED_K:]

    mesh = plsc.VectorSubcoreMesh(core_axis_name="c", subcore_axis_name="s")
    sc_params = pltpu.CompilerParams(needs_layout_passes=False)

    p = _matvec(W.T, wu)

    run_p = pl.kernel(
        _gather_p_body,
        mesh=mesh,
        compiler_params=sc_params,
        out_type=jax.ShapeDtypeStruct((BATCH,), jnp.float32),
        scratch_types=[
            pltpu.VMEM((B_PER_W,), jnp.int32),
            pltpu.VMEM((8 * B_PER_W,), jnp.float32),
            pltpu.VMEM((B_PER_W,), jnp.float32),
            pltpu.SemaphoreType.DMA,
        ],
    )
    pu = run_p(uidx, p)

    q = _matvec(H.T, wv)

    wb = jnp.concatenate(
        [lin_b.reshape(-1), jnp.zeros((15,), jnp.float32)]).astype(jnp.float32)

    run_q = pl.kernel(
        _gather_q_body,
        mesh=mesh,
        compiler_params=sc_params,
        out_type=jax.ShapeDtypeStruct((BATCH,), jnp.float32),
        scratch_types=[
            pltpu.VMEM((B_PER_W,), jnp.int32),
            pltpu.VMEM((8 * B_PER_W,), jnp.float32),
            pltpu.VMEM((B_PER_W,), jnp.float32),
            pltpu.VMEM((16,), jnp.float32),
            pltpu.VMEM((B_PER_W,), jnp.float32),
            pltpu.SemaphoreType.DMA,
        ],
    )
    return run_q(iidx, q, pu, wb)


# speedup vs baseline: 1.1200x; 1.1200x over previous
import functools

import jax
import jax.numpy as jnp
from jax import lax
from jax.experimental import pallas as pl
from jax.experimental.pallas import tpu as pltpu
from jax.experimental.pallas import tpu_sc as plsc

EMBED_K = 32
BATCH = 16384
NROWS = 1000000
NC = 2
NS = 16
LANES = 16
NW = NC * NS
B_PER_W = BATCH // NW
GROUPS = B_PER_W // LANES

BLK = 40960
NBLK = (NROWS + BLK - 1) // BLK


def _matvec_body(lw_ref, wt_ref, ht_ref, p_ref, q_ref):
    wu = lw_ref[:, :EMBED_K]
    wv = lw_ref[:, EMBED_K:]
    p = jax.lax.dot_general(wu, wt_ref[...], (((1,), (0,)), ((), ())),
                            preferred_element_type=jnp.float32)
    q = jax.lax.dot_general(wv, ht_ref[...], (((1,), (0,)), ((), ())),
                            preferred_element_type=jnp.float32)
    p_ref[...] = p[0]
    q_ref[...] = q[0]


def _gather_body(uidx_hbm, iidx_hbm, p_hbm, q_hbm, wb_hbm, out_hbm,
                 uidx_v, iidx_v, pu_v, qv_v, wb_v, out_v, sem):
    wid = lax.axis_index("s") * NC + lax.axis_index("c")
    base = wid * B_PER_W

    pltpu.sync_copy(uidx_hbm.at[pl.ds(base, B_PER_W)], uidx_v)
    pltpu.sync_copy(iidx_hbm.at[pl.ds(base, B_PER_W)], iidx_v)
    pltpu.sync_copy(wb_hbm, wb_v)

    def fire(g, carry):
        uvec = uidx_v[pl.ds(g * LANES, LANES)]
        ivec = iidx_v[pl.ds(g * LANES, LANES)]
        for j in range(LANES):
            r = g * LANES + j
            ua = pl.multiple_of((uvec[j] >> 3) << 3, 8)
            ia = pl.multiple_of((ivec[j] >> 3) << 3, 8)
            pltpu.async_copy(p_hbm.at[pl.ds(ua, 8)],
                             pu_v.at[pl.ds(r * 8, 8)], sem)
            pltpu.async_copy(q_hbm.at[pl.ds(ia, 8)],
                             qv_v.at[pl.ds(r * 8, 8)], sem)
        return carry

    lax.fori_loop(0, GROUPS, fire, 0, unroll=False)

    pltpu.make_async_copy(p_hbm.at[pl.ds(0, 8 * B_PER_W)], pu_v, sem).wait()
    pltpu.make_async_copy(q_hbm.at[pl.ds(0, 8 * B_PER_W)], qv_v, sem).wait()

    bias = wb_v[pl.ds(0, LANES)][0]
    lane8 = lax.iota(jnp.int32, LANES) * 8

    def group(g, carry):
        uoff = uidx_v[pl.ds(g * LANES, LANES)] & 7
        ioff = iidx_v[pl.ds(g * LANES, LANES)] & 7
        addr0 = g * (LANES * 8) + lane8
        zu = plsc.load_gather(pu_v, [addr0 + uoff])
        zv = plsc.load_gather(qv_v, [addr0 + ioff])
        z = zu + zv + bias
        out_v[pl.ds(g * LANES, LANES)] = 1.0 / (1.0 + jnp.exp(-z))
        return carry

    lax.fori_loop(0, GROUPS, group, 0, unroll=False)

    pltpu.sync_copy(out_v, out_hbm.at[pl.ds(base, B_PER_W)])


@functools.partial(jax.jit, static_argnames=())
def kernel(x, W, H, lin_w, lin_b):
    uidx = x[:, 0].astype(jnp.int32)
    iidx = x[:, 1].astype(jnp.int32)
    wt = W.T
    ht = H.T

    p, q = pl.pallas_call(
        _matvec_body,
        grid=(NBLK,),
        in_specs=[
            pl.BlockSpec((1, 2 * EMBED_K), lambda b: (0, 0)),
            pl.BlockSpec((EMBED_K, BLK), lambda b: (0, b)),
            pl.BlockSpec((EMBED_K, BLK), lambda b: (0, b)),
        ],
        out_specs=[
            pl.BlockSpec((BLK,), lambda b: (b,)),
            pl.BlockSpec((BLK,), lambda b: (b,)),
        ],
        out_shape=[
            jax.ShapeDtypeStruct((NROWS,), jnp.float32),
            jax.ShapeDtypeStruct((NROWS,), jnp.float32),
        ],
    )(lin_w, wt, ht)

    wb = jnp.concatenate(
        [lin_b.reshape(-1), jnp.zeros((15,), jnp.float32)]).astype(jnp.float32)

    mesh = plsc.VectorSubcoreMesh(core_axis_name="c", subcore_axis_name="s")
    run = pl.kernel(
        _gather_body,
        mesh=mesh,
        compiler_params=pltpu.CompilerParams(needs_layout_passes=False),
        out_type=jax.ShapeDtypeStruct((BATCH,), jnp.float32),
        scratch_types=[
            pltpu.VMEM((B_PER_W,), jnp.int32),
            pltpu.VMEM((B_PER_W,), jnp.int32),
            pltpu.VMEM((8 * B_PER_W,), jnp.float32),
            pltpu.VMEM((8 * B_PER_W,), jnp.float32),
            pltpu.VMEM((16,), jnp.float32),
            pltpu.VMEM((B_PER_W,), jnp.float32),
            pltpu.SemaphoreType.DMA,
        ],
    )
    return run(uidx, iidx, p, q, wb)


# speedup vs baseline: 1.1233x; 1.0029x over previous
import functools

import jax
import jax.numpy as jnp
from jax import lax
from jax.experimental import pallas as pl
from jax.experimental.pallas import tpu as pltpu
from jax.experimental.pallas import tpu_sc as plsc

EMBED_K = 32
BATCH = 16384
NROWS = 1000000
NC = 2
NS = 16
LANES = 16
NW = NC * NS
B_PER_W = BATCH // NW
GROUPS = B_PER_W // LANES

BLK = 32768
NBLK = (NROWS + BLK - 1) // BLK


def _matvec_body(lw_ref, wt_ref, ht_ref, p_ref, q_ref):
    wu = lw_ref[:, :EMBED_K]
    wv = lw_ref[:, EMBED_K:]
    p = jax.lax.dot_general(wu, wt_ref[...], (((1,), (0,)), ((), ())),
                            preferred_element_type=jnp.float32)
    q = jax.lax.dot_general(wv, ht_ref[...], (((1,), (0,)), ((), ())),
                            preferred_element_type=jnp.float32)
    p_ref[...] = p[0]
    q_ref[...] = q[0]


def _gather_body(uidx_hbm, iidx_hbm, p_hbm, q_hbm, wb_hbm, out_hbm,
                 uidx_v, iidx_v, pu_v, qv_v, wb_v, out_v, sem):
    wid = lax.axis_index("s") * NC + lax.axis_index("c")
    base = wid * B_PER_W

    pltpu.sync_copy(uidx_hbm.at[pl.ds(base, B_PER_W)], uidx_v)
    pltpu.sync_copy(iidx_hbm.at[pl.ds(base, B_PER_W)], iidx_v)
    pltpu.sync_copy(wb_hbm, wb_v)

    def fire(g, carry):
        uvec = uidx_v[pl.ds(g * LANES, LANES)]
        ivec = iidx_v[pl.ds(g * LANES, LANES)]
        for j in range(LANES):
            r = g * LANES + j
            ua = pl.multiple_of((uvec[j] >> 3) << 3, 8)
            ia = pl.multiple_of((ivec[j] >> 3) << 3, 8)
            pltpu.async_copy(p_hbm.at[pl.ds(ua, 8)],
                             pu_v.at[pl.ds(r * 8, 8)], sem)
            pltpu.async_copy(q_hbm.at[pl.ds(ia, 8)],
                             qv_v.at[pl.ds(r * 8, 8)], sem)
        return carry

    lax.fori_loop(0, GROUPS, fire, 0, unroll=False)

    pltpu.make_async_copy(p_hbm.at[pl.ds(0, 8 * B_PER_W)], pu_v, sem).wait()
    pltpu.make_async_copy(q_hbm.at[pl.ds(0, 8 * B_PER_W)], qv_v, sem).wait()

    bias = wb_v[pl.ds(0, LANES)][0]
    lane8 = lax.iota(jnp.int32, LANES) * 8

    def group(g, carry):
        uoff = uidx_v[pl.ds(g * LANES, LANES)] & 7
        ioff = iidx_v[pl.ds(g * LANES, LANES)] & 7
        addr0 = g * (LANES * 8) + lane8
        zu = plsc.load_gather(pu_v, [addr0 + uoff])
        zv = plsc.load_gather(qv_v, [addr0 + ioff])
        z = zu + zv + bias
        out_v[pl.ds(g * LANES, LANES)] = 1.0 / (1.0 + jnp.exp(-z))
        return carry

    lax.fori_loop(0, GROUPS, group, 0, unroll=False)

    pltpu.sync_copy(out_v, out_hbm.at[pl.ds(base, B_PER_W)])


@functools.partial(jax.jit, static_argnames=())
def kernel(x, W, H, lin_w, lin_b):
    uidx = x[:, 0].astype(jnp.int32)
    iidx = x[:, 1].astype(jnp.int32)
    wt = W.T
    ht = H.T

    p, q = pl.pallas_call(
        _matvec_body,
        grid=(NBLK,),
        in_specs=[
            pl.BlockSpec((1, 2 * EMBED_K), lambda b: (0, 0)),
            pl.BlockSpec((EMBED_K, BLK), lambda b: (0, b)),
            pl.BlockSpec((EMBED_K, BLK), lambda b: (0, b)),
        ],
        out_specs=[
            pl.BlockSpec((BLK,), lambda b: (b,)),
            pl.BlockSpec((BLK,), lambda b: (b,)),
        ],
        out_shape=[
            jax.ShapeDtypeStruct((NROWS,), jnp.float32),
            jax.ShapeDtypeStruct((NROWS,), jnp.float32),
        ],
    )(lin_w, wt, ht)

    wb = jnp.concatenate(
        [lin_b.reshape(-1), jnp.zeros((15,), jnp.float32)]).astype(jnp.float32)

    mesh = plsc.VectorSubcoreMesh(core_axis_name="c", subcore_axis_name="s")
    run = pl.kernel(
        _gather_body,
        mesh=mesh,
        compiler_params=pltpu.CompilerParams(needs_layout_passes=False),
        out_type=jax.ShapeDtypeStruct((BATCH,), jnp.float32),
        scratch_types=[
            pltpu.VMEM((B_PER_W,), jnp.int32),
            pltpu.VMEM((B_PER_W,), jnp.int32),
            pltpu.VMEM((8 * B_PER_W,), jnp.float32),
            pltpu.VMEM((8 * B_PER_W,), jnp.float32),
            pltpu.VMEM((16,), jnp.float32),
            pltpu.VMEM((B_PER_W,), jnp.float32),
            pltpu.SemaphoreType.DMA,
        ],
    )
    return run(uidx, iidx, p, q, wb)


# speedup vs baseline: 1.1300x; 1.0060x over previous
import functools

import jax
import jax.numpy as jnp
from jax import lax
from jax.experimental import pallas as pl
from jax.experimental.pallas import tpu as pltpu
from jax.experimental.pallas import tpu_sc as plsc

EMBED_K = 32
BATCH = 16384
NROWS = 1000000
NC = 2
NS = 16
LANES = 16
NW = NC * NS
B_PER_W = BATCH // NW
GROUPS = B_PER_W // LANES

BLK = 32768
NBLK = (NROWS + BLK - 1) // BLK


def _matvec_body(lw_ref, wt_ref, ht_ref, p_ref, q_ref):
    wu = lw_ref[:, :EMBED_K]
    wv = lw_ref[:, EMBED_K:]
    p = jax.lax.dot_general(wu, wt_ref[...], (((1,), (0,)), ((), ())),
                            preferred_element_type=jnp.float32)
    q = jax.lax.dot_general(wv, ht_ref[...], (((1,), (0,)), ((), ())),
                            preferred_element_type=jnp.float32)
    p_ref[...] = p[0]
    q_ref[...] = q[0]


def _gather_body(uidx_hbm, iidx_hbm, p_hbm, q_hbm, wb_hbm, out_hbm,
                 uidx_v, iidx_v, pu_v, qv_v, wb_v, out_v, sem):
    wid = lax.axis_index("s") * NC + lax.axis_index("c")
    base = wid * B_PER_W

    c1 = pltpu.async_copy(uidx_hbm.at[pl.ds(base, B_PER_W)], uidx_v, sem)
    c2 = pltpu.async_copy(iidx_hbm.at[pl.ds(base, B_PER_W)], iidx_v, sem)
    c3 = pltpu.async_copy(wb_hbm, wb_v, sem)
    c1.wait()
    c2.wait()
    c3.wait()

    def fire(g, carry):
        uvec = uidx_v[pl.ds(g * LANES, LANES)]
        ivec = iidx_v[pl.ds(g * LANES, LANES)]
        for j in range(LANES):
            r = g * LANES + j
            ua = pl.multiple_of((uvec[j] >> 3) << 3, 8)
            ia = pl.multiple_of((ivec[j] >> 3) << 3, 8)
            pltpu.async_copy(p_hbm.at[pl.ds(ua, 8)],
                             pu_v.at[pl.ds(r * 8, 8)], sem)
            pltpu.async_copy(q_hbm.at[pl.ds(ia, 8)],
                             qv_v.at[pl.ds(r * 8, 8)], sem)
        return carry

    lax.fori_loop(0, GROUPS, fire, 0, unroll=False)

    pltpu.make_async_copy(p_hbm.at[pl.ds(0, 8 * B_PER_W)], pu_v, sem).wait()
    pltpu.make_async_copy(q_hbm.at[pl.ds(0, 8 * B_PER_W)], qv_v, sem).wait()

    bias = wb_v[pl.ds(0, LANES)][0]
    lane8 = lax.iota(jnp.int32, LANES) * 8

    def group(g, carry):
        uoff = uidx_v[pl.ds(g * LANES, LANES)] & 7
        ioff = iidx_v[pl.ds(g * LANES, LANES)] & 7
        addr0 = g * (LANES * 8) + lane8
        zu = plsc.load_gather(pu_v, [addr0 + uoff])
        zv = plsc.load_gather(qv_v, [addr0 + ioff])
        z = zu + zv + bias
        out_v[pl.ds(g * LANES, LANES)] = 1.0 / (1.0 + jnp.exp(-z))
        return carry

    lax.fori_loop(0, GROUPS, group, 0, unroll=False)

    pltpu.sync_copy(out_v, out_hbm.at[pl.ds(base, B_PER_W)])


@functools.partial(jax.jit, static_argnames=())
def kernel(x, W, H, lin_w, lin_b):
    uidx = x[:, 0].astype(jnp.int32)
    iidx = x[:, 1].astype(jnp.int32)
    wt = W.T
    ht = H.T

    p, q = pl.pallas_call(
        _matvec_body,
        grid=(NBLK,),
        in_specs=[
            pl.BlockSpec((1, 2 * EMBED_K), lambda b: (0, 0)),
            pl.BlockSpec((EMBED_K, BLK), lambda b: (0, b)),
            pl.BlockSpec((EMBED_K, BLK), lambda b: (0, b)),
        ],
        out_specs=[
            pl.BlockSpec((BLK,), lambda b: (b,)),
            pl.BlockSpec((BLK,), lambda b: (b,)),
        ],
        out_shape=[
            jax.ShapeDtypeStruct((NROWS,), jnp.float32),
            jax.ShapeDtypeStruct((NROWS,), jnp.float32),
        ],
    )(lin_w, wt, ht)

    wb = jnp.concatenate(
        [lin_b.reshape(-1), jnp.zeros((15,), jnp.float32)]).astype(jnp.float32)

    mesh = plsc.VectorSubcoreMesh(core_axis_name="c", subcore_axis_name="s")
    run = pl.kernel(
        _gather_body,
        mesh=mesh,
        compiler_params=pltpu.CompilerParams(needs_layout_passes=False),
        out_type=jax.ShapeDtypeStruct((BATCH,), jnp.float32),
        scratch_types=[
            pltpu.VMEM((B_PER_W,), jnp.int32),
            pltpu.VMEM((B_PER_W,), jnp.int32),
            pltpu.VMEM((8 * B_PER_W,), jnp.float32),
            pltpu.VMEM((8 * B_PER_W,), jnp.float32),
            pltpu.VMEM((16,), jnp.float32),
            pltpu.VMEM((B_PER_W,), jnp.float32),
            pltpu.SemaphoreType.DMA,
        ],
    )
    return run(uidx, iidx, p, q, wb)
